# baseline (speedup 1.0000x reference)
import functools

import jax
import jax.numpy as jnp
from jax import lax
from jax.experimental import pallas as pl
from jax.experimental.pallas import tpu as pltpu
from jax.experimental.pallas import tpu_sc as plsc

_PADF = 32


@functools.lru_cache(maxsize=None)
def _build(B, F, D):
    info = plsc.get_sparse_core_info()
    NC, NS = info.num_cores, info.num_subcores
    NW = NC * NS
    assert B % NW == 0
    b_per_w = B // NW
    mesh = plsc.VectorSubcoreMesh(core_axis_name="c", subcore_axis_name="s")

    half = b_per_w // 2

    @functools.partial(
        pl.kernel,
        mesh=mesh,
        out_type=jax.ShapeDtypeStruct((B, F, D), jnp.float32),
        scratch_types=[
            pltpu.VMEM((half, _PADF), jnp.int32),
            pltpu.VMEM((half, _PADF, D), jnp.float32),
            pltpu.SemaphoreType.DMA,
        ],
        compiler_params=pltpu.CompilerParams(use_tc_tiling_on_sc=False),
    )
    def k(idx_hbm, table_hbm, out_hbm, idx_v, rows_v, sem):
        wid = lax.axis_index("s") * NC + lax.axis_index("c")
        for h in range(2):
            base = wid * b_per_w + h * half
            pltpu.sync_copy(idx_hbm.at[pl.ds(base, half), :], idx_v)
            copies = [
                pltpu.async_copy(table_hbm.at[idx_v.at[j]], rows_v.at[j], sem)
                for j in range(half)
            ]
            for c in copies:
                c.wait()
            pltpu.sync_copy(
                rows_v.at[:, pl.ds(0, F), :], out_hbm.at[pl.ds(base, half)]
            )

    return k


def _pad_body(x_ref, o_ref):
    o_ref[...] = jnp.pad(
        x_ref[...], ((0, 0), (0, o_ref.shape[1] - x_ref.shape[1]))
    )


@functools.lru_cache(maxsize=None)
def _pad(B, F):
    return pl.pallas_call(
        _pad_body,
        out_shape=jax.ShapeDtypeStruct((B, _PADF), jnp.int32),
    )


def kernel(x, weight):
    B, F = x.shape
    D = weight.shape[1]
    idx = _pad(B, F)(x.astype(jnp.int32))
    return _build(B, F, D)(idx, weight)
